# baseline (speedup 1.0000x reference)
import jax
import jax.numpy as jnp
from jax.experimental import pallas as pl


_BLOCK_S = 512


def _make_add_kernel(batch):
    def _add_pos_kernel(x_ref, pos_ref, out_ref):
        p = pos_ref[...]
        out_ref[...] = x_ref[...] + jnp.concatenate([p] * batch, axis=1)
    return _add_pos_kernel


def kernel(x, pos_table):
    seq_len, batch, d_model = x.shape
    x2 = x.reshape(seq_len, batch * d_model)
    grid = (seq_len // _BLOCK_S,)
    out2 = pl.pallas_call(
        _make_add_kernel(batch),
        grid=grid,
        in_specs=[
            pl.BlockSpec((_BLOCK_S, batch * d_model), lambda i: (i, 0)),
            pl.BlockSpec((_BLOCK_S, d_model), lambda i: (i, 0)),
        ],
        out_specs=pl.BlockSpec((_BLOCK_S, batch * d_model), lambda i: (i, 0)),
        out_shape=jax.ShapeDtypeStruct((seq_len, batch * d_model), x.dtype),
    )(x2, pos_table[:seq_len])
    return out2.reshape(seq_len, batch, d_model)


# speedup vs baseline: 3.8350x; 3.8350x over previous
import jax
import jax.numpy as jnp
from jax.experimental import pallas as pl


_BLOCK_S = 512


def _add_pos_kernel(x_ref, pos_ref, out_ref):
    out_ref[...] = x_ref[...] + pos_ref[...][:, None, :]


def kernel(x, pos_table):
    seq_len, batch, d_model = x.shape
    grid = (seq_len // _BLOCK_S,)
    return pl.pallas_call(
        _add_pos_kernel,
        grid=grid,
        in_specs=[
            pl.BlockSpec((_BLOCK_S, batch, d_model), lambda i: (i, 0, 0)),
            pl.BlockSpec((_BLOCK_S, d_model), lambda i: (i, 0)),
        ],
        out_specs=pl.BlockSpec((_BLOCK_S, batch, d_model), lambda i: (i, 0, 0)),
        out_shape=jax.ShapeDtypeStruct((seq_len, batch, d_model), x.dtype),
    )(x, pos_table[:seq_len])
